# baseline (speedup 1.0000x reference)
import jax
import jax.numpy as jnp
from jax import lax
from jax.experimental import pallas as pl
from jax.experimental.pallas import tpu as pltpu
from jax.experimental.pallas import tpu_sc as plsc

_N = 307
_H = 128
_L = 16
_NC = 2
_NS = 16
_NW = _NC * _NS
_NPAD = 312
_EP = 992


def _proj_body(x_ref, wl_ref, wr_ref, xl_ref, xr_ref):
    x = x_ref[...]
    dn = (((1,), (1,)), ((), ()))
    xl_ref[...] = lax.dot_general(x, wl_ref[...], dn,
                                  preferred_element_type=jnp.float32)
    xr_ref[...] = lax.dot_general(x, wr_ref[...], dn,
                                  preferred_element_type=jnp.float32)


def _sc_body(xl_hbm, xr_hbm, src_hbm, dst_hbm, att_hbm, bias_hbm, out_hbm,
             xl_v, xr_v, out_v, src_v, dst_v, e_v, den_v, att_v, bias_v, sem):
    wid = lax.axis_index("s") * _NC + lax.axis_index("c")
    c_small = [
        pltpu.async_copy(src_hbm, src_v, sem),
        pltpu.async_copy(dst_hbm, dst_v, sem),
        pltpu.async_copy(att_hbm, att_v, sem),
        pltpu.async_copy(bias_hbm, bias_v, sem),
    ]
    base0 = wid * (_N * _H)
    base1 = (wid + _NW) * (_N * _H)
    c_xl = pltpu.async_copy(xl_hbm.at[pl.ds(base0, _N * _H)],
                            xl_v.at[pl.ds(0, _N * _H)], sem)
    c_xr = pltpu.async_copy(xr_hbm.at[pl.ds(base0, _N * _H)],
                            xr_v.at[pl.ds(0, _N * _H)], sem)

    zero16 = jnp.zeros((_L,), jnp.float32)
    iota16 = lax.iota(jnp.int32, _L)

    def zpad(i, c):
        xl_v[pl.ds(_N * _H + i * _L, _L)] = zero16
        xr_v[pl.ds(_N * _H + i * _L, _L)] = zero16
        return c
    lax.fori_loop(0, (_NPAD - _N) * _H // _L, zpad, 0)

    for c in c_small:
        c.wait()
    bias_chunks = [bias_v[pl.ds(fc * _L, _L)] for fc in range(_H // _L)]

    def binit(row, c):
        for fc in range(_H // _L):
            out_v[pl.ds(row * _H + fc * _L, _L)] = bias_chunks[fc]
        return c

    def dzero(i, c):
        den_v[pl.ds(i * _L, _L)] = zero16
        return c

    def run_passes():
        def score_chunk(c, gmax):
            e0 = c * (2 * _L)
            sa16 = src_v[pl.ds(e0, _L)]
            da16 = dst_v[pl.ds(e0, _L)]
            sb16 = src_v[pl.ds(e0 + _L, _L)]
            db16 = dst_v[pl.ds(e0 + _L, _L)]
            sba = sa16 * _H
            dba = da16 * _H
            sbb = sb16 * _H
            dbb = db16 * _H
            def fblock(fo, accs):
                accs = list(accs)
                fb = fo * _L
                for fi in range(_L):
                    rot = iota16 ^ (fb + fi)
                    attf = att_v[pl.ds((fb + fi) * _L, _L)]
                    sla = plsc.load_gather(xl_v, [sba + rot])
                    dla = plsc.load_gather(xr_v, [dba + rot])
                    slb = plsc.load_gather(xl_v, [sbb + rot])
                    dlb = plsc.load_gather(xr_v, [dbb + rot])
                    ma = sla + dla
                    mb = slb + dlb
                    ha = jnp.where(ma >= 0, ma, jnp.float32(0.2) * ma)
                    hb = jnp.where(mb >= 0, mb, jnp.float32(0.2) * mb)
                    accs[fi % 2] = accs[fi % 2] + attf * ha
                    accs[2 + fi % 2] = accs[2 + fi % 2] + attf * hb
                return tuple(accs)
            accs = lax.fori_loop(0, _H // _L, fblock,
                                 (zero16, zero16, zero16, zero16))
            acc_a = accs[0] + accs[1]
            acc_b = accs[2] + accs[3]
            e_v[pl.ds(e0, _L)] = acc_a
            e_v[pl.ds(e0 + _L, _L)] = acc_b
            return jnp.maximum(gmax, jnp.maximum(acc_a, acc_b))
        gmax16 = lax.fori_loop(0, _EP // (2 * _L), score_chunk,
                               jnp.full((_L,), -1e30, jnp.float32))
        gmax = jnp.max(gmax16)

        def den_chunk(c, carry):
            dst16 = dst_v[pl.ds(c * _L, _L)]
            ex = jnp.exp(e_v[pl.ds(c * _L, _L)] - gmax)
            e_v[pl.ds(c * _L, _L)] = ex
            plsc.addupdate_scatter(den_v, [dst16], ex)
            return carry
        lax.fori_loop(0, _EP // _L, den_chunk, 0)

        def agg_chunk(c, carry):
            src16 = src_v[pl.ds(c * _L, _L)]
            dst16 = dst_v[pl.ds(c * _L, _L)]
            sb = src16 * _H
            db = dst16 * _H
            ex = e_v[pl.ds(c * _L, _L)]
            dn = plsc.load_gather(den_v, [dst16])
            alpha = ex / (dn + jnp.float32(1e-16))
            def ablock(fo, c2):
                fb = fo * _L
                for fi in range(_L):
                    rot = iota16 ^ (fb + fi)
                    xv = plsc.load_gather(xl_v, [sb + rot])
                    plsc.addupdate_scatter(out_v, [db + rot], alpha * xv)
                return c2
            lax.fori_loop(0, _H // _L, ablock, 0)
            return carry
        lax.fori_loop(0, _EP // _L, agg_chunk, 0)

    lax.fori_loop(0, _NPAD, binit, 0)
    lax.fori_loop(0, 320 // _L, dzero, 0)
    c_xl.wait()
    c_xr.wait()
    run_passes()

    c_out = pltpu.async_copy(out_v.at[pl.ds(0, _N * _H)],
                             out_hbm.at[pl.ds(base0, _N * _H)], sem)
    c_xl = pltpu.async_copy(xl_hbm.at[pl.ds(base1, _N * _H)],
                            xl_v.at[pl.ds(0, _N * _H)], sem)
    c_xr = pltpu.async_copy(xr_hbm.at[pl.ds(base1, _N * _H)],
                            xr_v.at[pl.ds(0, _N * _H)], sem)
    lax.fori_loop(0, 320 // _L, dzero, 0)
    c_out.wait()
    lax.fori_loop(0, _NPAD, binit, 0)
    c_xl.wait()
    c_xr.wait()
    run_passes()
    pltpu.sync_copy(out_v.at[pl.ds(0, _N * _H)],
                    out_hbm.at[pl.ds(base1, _N * _H)])


def kernel(t, z, edge_index, Wl, Wr, att, bias):
    h = z.shape[1]
    n = _N
    b = z.shape[0] // n
    e = edge_index.shape[1]
    et = e + n
    loop = jnp.arange(n, dtype=jnp.int32)
    pad = jnp.full((_EP - et,), n, jnp.int32)
    src = jnp.concatenate([edge_index[0].astype(jnp.int32), loop, pad])
    dst = jnp.concatenate([edge_index[1].astype(jnp.int32), loop, pad])

    rows = b * n
    nch = 8
    blk = rows // nch
    xl, xr = pl.pallas_call(
        _proj_body,
        grid=(nch,),
        in_specs=[
            pl.BlockSpec((blk, h), lambda i: (i, 0)),
            pl.BlockSpec((h, h), lambda i: (0, 0)),
            pl.BlockSpec((h, h), lambda i: (0, 0)),
        ],
        out_specs=[
            pl.BlockSpec((blk, h), lambda i: (i, 0)),
            pl.BlockSpec((blk, h), lambda i: (i, 0)),
        ],
        out_shape=[
            jax.ShapeDtypeStruct((rows, h), jnp.float32),
            jax.ShapeDtypeStruct((rows, h), jnp.float32),
        ],
    )(z, Wl, Wr)

    sc = pl.kernel(
        _sc_body,
        out_type=jax.ShapeDtypeStruct((rows * h,), jnp.float32),
        mesh=plsc.VectorSubcoreMesh(core_axis_name="c", subcore_axis_name="s",
                                    num_cores=_NC, num_subcores=_NS),
        compiler_params=pltpu.CompilerParams(needs_layout_passes=False),
        scratch_types=[
            pltpu.VMEM((_NPAD * _H,), jnp.float32),
            pltpu.VMEM((_NPAD * _H,), jnp.float32),
            pltpu.VMEM((_NPAD * _H,), jnp.float32),
            pltpu.VMEM((_EP,), jnp.int32),
            pltpu.VMEM((_EP,), jnp.int32),
            pltpu.VMEM((_EP,), jnp.float32),
            pltpu.VMEM((320,), jnp.float32),
            pltpu.VMEM((_H * _L,), jnp.float32),
            pltpu.VMEM((h,), jnp.float32),
            pltpu.SemaphoreType.DMA,
        ],
    )
    rot_idx = jnp.arange(_H)[:, None] ^ jnp.arange(_L)[None, :]
    att_tab = att[rot_idx].reshape(-1)
    out = sc(xl.reshape(-1), xr.reshape(-1), src, dst, att_tab, bias)
    return out.reshape(rows, h, 1)


# speedup vs baseline: 1.2128x; 1.2128x over previous
import jax
import jax.numpy as jnp
from jax import lax
from jax.experimental import pallas as pl
from jax.experimental.pallas import tpu as pltpu
from jax.experimental.pallas import tpu_sc as plsc

_N = 307
_H = 128
_L = 16
_NC = 2
_NS = 16
_NW = _NC * _NS
_NPAD = 312
_EP = 992


def _proj_body(x_ref, wl_ref, wr_ref, xl_ref, xr_ref):
    x = x_ref[...]
    dn = (((1,), (1,)), ((), ()))
    xl_ref[...] = lax.dot_general(x, wl_ref[...], dn,
                                  preferred_element_type=jnp.float32)
    xr_ref[...] = lax.dot_general(x, wr_ref[...], dn,
                                  preferred_element_type=jnp.float32)


def _sc_body(xl_hbm, xr_hbm, src_hbm, dst_hbm, att_hbm, bias_hbm, out_hbm,
             xl_v, xr_v, out_v, src_v, dst_v, e_v, den_v, att_v, bias_v, sem):
    wid = lax.axis_index("s") * _NC + lax.axis_index("c")
    c_small = [
        pltpu.async_copy(src_hbm, src_v, sem),
        pltpu.async_copy(dst_hbm, dst_v, sem),
        pltpu.async_copy(att_hbm, att_v, sem),
        pltpu.async_copy(bias_hbm, bias_v, sem),
    ]
    base0 = wid * (_N * _H)
    base1 = (wid + _NW) * (_N * _H)
    c_xl = pltpu.async_copy(xl_hbm.at[pl.ds(base0, _N * _H)],
                            xl_v.at[pl.ds(0, _N * _H)], sem)
    c_xr = pltpu.async_copy(xr_hbm.at[pl.ds(base0, _N * _H)],
                            xr_v.at[pl.ds(0, _N * _H)], sem)

    zero16 = jnp.zeros((_L,), jnp.float32)
    iota16 = lax.iota(jnp.int32, _L)

    def zpad(i, c):
        xl_v[pl.ds(_N * _H + i * _L, _L)] = zero16
        xr_v[pl.ds(_N * _H + i * _L, _L)] = zero16
        return c
    lax.fori_loop(0, (_NPAD - _N) * _H // _L, zpad, 0)

    for c in c_small:
        c.wait()
    bias_chunks = [bias_v[pl.ds(fc * _L, _L)] for fc in range(_H // _L)]

    def binit(row, c):
        for fc in range(_H // _L):
            out_v[pl.ds(row * _H + fc * _L, _L)] = bias_chunks[fc]
        return c

    def dzero(i, c):
        den_v[pl.ds(i * _L, _L)] = zero16
        return c

    def run_passes():
        def score_chunk(c, gmax):
            src16 = src_v[pl.ds(c * _L, _L)]
            dst16 = dst_v[pl.ds(c * _L, _L)]
            sb = src16 * _H
            db = dst16 * _H
            def fblock(fo, accs):
                accs = list(accs)
                fb = fo * _L
                for fi in range(_L):
                    rot = iota16 ^ (fb + fi)
                    attf = att_v[pl.ds((fb + fi) * _L, _L)]
                    sl = plsc.load_gather(xl_v, [sb + rot])
                    dl = plsc.load_gather(xr_v, [db + rot])
                    m = sl + dl
                    hh = jnp.where(m >= 0, m, jnp.float32(0.2) * m)
                    accs[fi % 4] = accs[fi % 4] + attf * hh
                return tuple(accs)
            accs = lax.fori_loop(0, _H // _L, fblock,
                                 (zero16, zero16, zero16, zero16))
            acc = (accs[0] + accs[1]) + (accs[2] + accs[3])
            e_v[pl.ds(c * _L, _L)] = acc
            return jnp.maximum(gmax, acc)
        gmax16 = lax.fori_loop(0, _EP // _L, score_chunk,
                               jnp.full((_L,), -1e30, jnp.float32))
        gmax = jnp.max(gmax16)

        def den_chunk(c, carry):
            dst16 = dst_v[pl.ds(c * _L, _L)]
            ex = jnp.exp(e_v[pl.ds(c * _L, _L)] - gmax)
            e_v[pl.ds(c * _L, _L)] = ex
            plsc.addupdate_scatter(den_v, [dst16], ex)
            return carry
        lax.fori_loop(0, _EP // _L, den_chunk, 0)

        def agg_chunk(c, carry):
            src16 = src_v[pl.ds(c * _L, _L)]
            dst16 = dst_v[pl.ds(c * _L, _L)]
            sb = src16 * _H
            db = dst16 * _H
            ex = e_v[pl.ds(c * _L, _L)]
            dn = plsc.load_gather(den_v, [dst16])
            alpha = ex / (dn + jnp.float32(1e-16))
            def ablock(fo, c2):
                fb = fo * _L
                for fi in range(_L):
                    rot = iota16 ^ (fb + fi)
                    xv = plsc.load_gather(xl_v, [sb + rot])
                    plsc.addupdate_scatter(out_v, [db + rot], alpha * xv)
                return c2
            lax.fori_loop(0, _H // _L, ablock, 0)
            return carry
        lax.fori_loop(0, _EP // _L, agg_chunk, 0)

    lax.fori_loop(0, _NPAD, binit, 0)
    lax.fori_loop(0, 320 // _L, dzero, 0)
    c_xl.wait()
    c_xr.wait()
    run_passes()

    c_out = pltpu.async_copy(out_v.at[pl.ds(0, _N * _H)],
                             out_hbm.at[pl.ds(base0, _N * _H)], sem)
    c_xl = pltpu.async_copy(xl_hbm.at[pl.ds(base1, _N * _H)],
                            xl_v.at[pl.ds(0, _N * _H)], sem)
    c_xr = pltpu.async_copy(xr_hbm.at[pl.ds(base1, _N * _H)],
                            xr_v.at[pl.ds(0, _N * _H)], sem)
    lax.fori_loop(0, 320 // _L, dzero, 0)
    c_out.wait()
    lax.fori_loop(0, _NPAD, binit, 0)
    c_xl.wait()
    c_xr.wait()
    run_passes()
    pltpu.sync_copy(out_v.at[pl.ds(0, _N * _H)],
                    out_hbm.at[pl.ds(base1, _N * _H)])


def kernel(t, z, edge_index, Wl, Wr, att, bias):
    h = z.shape[1]
    n = _N
    b = z.shape[0] // n
    e = edge_index.shape[1]
    et = e + n
    loop = jnp.arange(n, dtype=jnp.int32)
    pad = jnp.full((_EP - et,), n, jnp.int32)
    src = jnp.concatenate([edge_index[0].astype(jnp.int32), loop, pad])
    dst = jnp.concatenate([edge_index[1].astype(jnp.int32), loop, pad])

    rows = b * n
    nch = 8
    blk = rows // nch
    xl, xr = pl.pallas_call(
        _proj_body,
        grid=(nch,),
        in_specs=[
            pl.BlockSpec((blk, h), lambda i: (i, 0)),
            pl.BlockSpec((h, h), lambda i: (0, 0)),
            pl.BlockSpec((h, h), lambda i: (0, 0)),
        ],
        out_specs=[
            pl.BlockSpec((blk, h), lambda i: (i, 0)),
            pl.BlockSpec((blk, h), lambda i: (i, 0)),
        ],
        out_shape=[
            jax.ShapeDtypeStruct((rows, h), jnp.float32),
            jax.ShapeDtypeStruct((rows, h), jnp.float32),
        ],
    )(z, Wl, Wr)

    sc = pl.kernel(
        _sc_body,
        out_type=jax.ShapeDtypeStruct((rows * h,), jnp.float32),
        mesh=plsc.VectorSubcoreMesh(core_axis_name="c", subcore_axis_name="s",
                                    num_cores=_NC, num_subcores=_NS),
        compiler_params=pltpu.CompilerParams(needs_layout_passes=False),
        scratch_types=[
            pltpu.VMEM((_NPAD * _H,), jnp.float32),
            pltpu.VMEM((_NPAD * _H,), jnp.float32),
            pltpu.VMEM((_NPAD * _H,), jnp.float32),
            pltpu.VMEM((_EP,), jnp.int32),
            pltpu.VMEM((_EP,), jnp.int32),
            pltpu.VMEM((_EP,), jnp.float32),
            pltpu.VMEM((320,), jnp.float32),
            pltpu.VMEM((_H * _L,), jnp.float32),
            pltpu.VMEM((h,), jnp.float32),
            pltpu.SemaphoreType.DMA,
        ],
    )
    rot_idx = jnp.arange(_H)[:, None] ^ jnp.arange(_L)[None, :]
    att_tab = att[rot_idx].reshape(-1)
    out = sc(xl.reshape(-1), xr.reshape(-1), src, dst, att_tab, bias)
    return out.reshape(rows, h, 1)


# speedup vs baseline: 1.2187x; 1.0048x over previous
import jax
import jax.numpy as jnp
from jax import lax
from jax.experimental import pallas as pl
from jax.experimental.pallas import tpu as pltpu
from jax.experimental.pallas import tpu_sc as plsc

_N = 307
_H = 128
_L = 16
_NC = 2
_NS = 16
_NW = _NC * _NS
_NPAD = 312
_EP = 992


def _proj_body(x_ref, wl_ref, wr_ref, xl_ref, xr_ref):
    x = x_ref[...]
    dn = (((1,), (1,)), ((), ()))
    xl_ref[...] = lax.dot_general(x, wl_ref[...], dn,
                                  preferred_element_type=jnp.float32)
    xr_ref[...] = lax.dot_general(x, wr_ref[...], dn,
                                  preferred_element_type=jnp.float32)


def _sc_body(xl_hbm, xr_hbm, src_hbm, dst_hbm, att_hbm, bias_hbm, out_hbm,
             xl_v, xr_v, out_v, src_v, dst_v, e_v, den_v, att_v, bias_v, sem):
    wid = lax.axis_index("s") * _NC + lax.axis_index("c")
    c_small = [
        pltpu.async_copy(src_hbm, src_v, sem),
        pltpu.async_copy(dst_hbm, dst_v, sem),
        pltpu.async_copy(att_hbm, att_v, sem),
        pltpu.async_copy(bias_hbm, bias_v, sem),
    ]
    base0 = wid * (_N * _H)
    base1 = (wid + _NW) * (_N * _H)
    c_xl = pltpu.async_copy(xl_hbm.at[pl.ds(base0, _N * _H)],
                            xl_v.at[pl.ds(0, _N * _H)], sem)
    c_xr = pltpu.async_copy(xr_hbm.at[pl.ds(base0, _N * _H)],
                            xr_v.at[pl.ds(0, _N * _H)], sem)

    zero16 = jnp.zeros((_L,), jnp.float32)
    iota16 = lax.iota(jnp.int32, _L)

    def zpad(i, c):
        xl_v[pl.ds(_N * _H + i * _L, _L)] = zero16
        xr_v[pl.ds(_N * _H + i * _L, _L)] = zero16
        return c
    lax.fori_loop(0, (_NPAD - _N) * _H // _L, zpad, 0)

    for c in c_small:
        c.wait()
    bias_chunks = [bias_v[pl.ds(fc * _L, _L)] for fc in range(_H // _L)]

    def binit(row, c):
        for fc in range(_H // _L):
            out_v[pl.ds(row * _H + fc * _L, _L)] = bias_chunks[fc]
        return c

    def dzero(i, c):
        den_v[pl.ds(i * _L, _L)] = zero16
        return c

    def run_passes():
        @plsc.parallel_loop(0, _EP // _L, carry=jnp.full((_L,), -1e30,
                                                         jnp.float32))
        def score_chunk(c, gmax):
            src16 = src_v[pl.ds(c * _L, _L)]
            dst16 = dst_v[pl.ds(c * _L, _L)]
            sb = src16 * _H
            db = dst16 * _H
            def fblock(fo, accs):
                accs = list(accs)
                fb = fo * _L
                for fi in range(_L):
                    rot = iota16 ^ (fb + fi)
                    attf = att_v[pl.ds((fb + fi) * _L, _L)]
                    sl = plsc.load_gather(xl_v, [sb + rot])
                    dl = plsc.load_gather(xr_v, [db + rot])
                    m = sl + dl
                    hh = jnp.where(m >= 0, m, jnp.float32(0.2) * m)
                    accs[fi % 4] = accs[fi % 4] + attf * hh
                return tuple(accs)
            accs = lax.fori_loop(0, _H // _L, fblock,
                                 (zero16, zero16, zero16, zero16))
            acc = (accs[0] + accs[1]) + (accs[2] + accs[3])
            e_v[pl.ds(c * _L, _L)] = acc
            return jnp.maximum(gmax, acc)
        gmax = jnp.max(score_chunk)

        @plsc.parallel_loop(0, _EP // _L, carry=jnp.int32(0))
        def den_chunk(c, carry):
            dst16 = dst_v[pl.ds(c * _L, _L)]
            ex = jnp.exp(e_v[pl.ds(c * _L, _L)] - gmax)
            e_v[pl.ds(c * _L, _L)] = ex
            plsc.addupdate_scatter(den_v, [dst16], ex)
            return carry

        @plsc.parallel_loop(0, _EP // _L, carry=jnp.int32(0))
        def agg_chunk(c, carry):
            src16 = src_v[pl.ds(c * _L, _L)]
            dst16 = dst_v[pl.ds(c * _L, _L)]
            sb = src16 * _H
            db = dst16 * _H
            ex = e_v[pl.ds(c * _L, _L)]
            dn = plsc.load_gather(den_v, [dst16])
            alpha = ex / (dn + jnp.float32(1e-16))
            def ablock(fo, c2):
                fb = fo * _L
                for fi in range(_L):
                    rot = iota16 ^ (fb + fi)
                    xv = plsc.load_gather(xl_v, [sb + rot])
                    plsc.addupdate_scatter(out_v, [db + rot], alpha * xv)
                return c2
            lax.fori_loop(0, _H // _L, ablock, 0)
            return carry

    lax.fori_loop(0, _NPAD, binit, 0)
    lax.fori_loop(0, 320 // _L, dzero, 0)
    c_xl.wait()
    c_xr.wait()
    run_passes()

    c_out = pltpu.async_copy(out_v.at[pl.ds(0, _N * _H)],
                             out_hbm.at[pl.ds(base0, _N * _H)], sem)
    c_xl = pltpu.async_copy(xl_hbm.at[pl.ds(base1, _N * _H)],
                            xl_v.at[pl.ds(0, _N * _H)], sem)
    c_xr = pltpu.async_copy(xr_hbm.at[pl.ds(base1, _N * _H)],
                            xr_v.at[pl.ds(0, _N * _H)], sem)
    lax.fori_loop(0, 320 // _L, dzero, 0)
    c_out.wait()
    lax.fori_loop(0, _NPAD, binit, 0)
    c_xl.wait()
    c_xr.wait()
    run_passes()
    pltpu.sync_copy(out_v.at[pl.ds(0, _N * _H)],
                    out_hbm.at[pl.ds(base1, _N * _H)])


def kernel(t, z, edge_index, Wl, Wr, att, bias):
    h = z.shape[1]
    n = _N
    b = z.shape[0] // n
    e = edge_index.shape[1]
    et = e + n
    loop = jnp.arange(n, dtype=jnp.int32)
    pad = jnp.full((_EP - et,), n, jnp.int32)
    src = jnp.concatenate([edge_index[0].astype(jnp.int32), loop, pad])
    dst = jnp.concatenate([edge_index[1].astype(jnp.int32), loop, pad])

    rows = b * n
    nch = 8
    blk = rows // nch
    xl, xr = pl.pallas_call(
        _proj_body,
        grid=(nch,),
        in_specs=[
            pl.BlockSpec((blk, h), lambda i: (i, 0)),
            pl.BlockSpec((h, h), lambda i: (0, 0)),
            pl.BlockSpec((h, h), lambda i: (0, 0)),
        ],
        out_specs=[
            pl.BlockSpec((blk, h), lambda i: (i, 0)),
            pl.BlockSpec((blk, h), lambda i: (i, 0)),
        ],
        out_shape=[
            jax.ShapeDtypeStruct((rows, h), jnp.float32),
            jax.ShapeDtypeStruct((rows, h), jnp.float32),
        ],
    )(z, Wl, Wr)

    sc = pl.kernel(
        _sc_body,
        out_type=jax.ShapeDtypeStruct((rows * h,), jnp.float32),
        mesh=plsc.VectorSubcoreMesh(core_axis_name="c", subcore_axis_name="s",
                                    num_cores=_NC, num_subcores=_NS),
        compiler_params=pltpu.CompilerParams(needs_layout_passes=False),
        scratch_types=[
            pltpu.VMEM((_NPAD * _H,), jnp.float32),
            pltpu.VMEM((_NPAD * _H,), jnp.float32),
            pltpu.VMEM((_NPAD * _H,), jnp.float32),
            pltpu.VMEM((_EP,), jnp.int32),
            pltpu.VMEM((_EP,), jnp.int32),
            pltpu.VMEM((_EP,), jnp.float32),
            pltpu.VMEM((320,), jnp.float32),
            pltpu.VMEM((_H * _L,), jnp.float32),
            pltpu.VMEM((h,), jnp.float32),
            pltpu.SemaphoreType.DMA,
        ],
    )
    rot_idx = jnp.arange(_H)[:, None] ^ jnp.arange(_L)[None, :]
    att_tab = att[rot_idx].reshape(-1)
    out = sc(xl.reshape(-1), xr.reshape(-1), src, dst, att_tab, bias)
    return out.reshape(rows, h, 1)


# speedup vs baseline: 1.2480x; 1.0241x over previous
import jax
import jax.numpy as jnp
from jax import lax
from jax.experimental import pallas as pl
from jax.experimental.pallas import tpu as pltpu
from jax.experimental.pallas import tpu_sc as plsc

_N = 307
_H = 128
_L = 16
_NC = 2
_NS = 16
_NW = _NC * _NS
_NPAD = 312
_EP = 992


def _proj_body(x_ref, wl_ref, wr_ref, xl_ref, xr_ref):
    x = x_ref[...]
    dn = (((1,), (1,)), ((), ()))
    xl_ref[...] = lax.dot_general(x, wl_ref[...], dn,
                                  preferred_element_type=jnp.float32)
    xr_ref[...] = lax.dot_general(x, wr_ref[...], dn,
                                  preferred_element_type=jnp.float32)


def _sc_body(xl_hbm, xr_hbm, src_hbm, dst_hbm, att_hbm, bias_hbm, out_hbm,
             xl_v, xr_v, out_v, src_v, dst_v, e_v, den_v, att_v, bias_v, sem):
    wid = lax.axis_index("s") * _NC + lax.axis_index("c")
    c_small = [
        pltpu.async_copy(src_hbm, src_v, sem),
        pltpu.async_copy(dst_hbm, dst_v, sem),
        pltpu.async_copy(att_hbm, att_v, sem),
        pltpu.async_copy(bias_hbm, bias_v, sem),
    ]
    base0 = wid * (_N * _H)
    base1 = (wid + _NW) * (_N * _H)
    c_xl = pltpu.async_copy(xl_hbm.at[pl.ds(base0, _N * _H)],
                            xl_v.at[pl.ds(0, _N * _H)], sem)
    c_xr = pltpu.async_copy(xr_hbm.at[pl.ds(base0, _N * _H)],
                            xr_v.at[pl.ds(0, _N * _H)], sem)

    zero16 = jnp.zeros((_L,), jnp.float32)
    iota16 = lax.iota(jnp.int32, _L)

    def zpad(i, c):
        xl_v[pl.ds(_N * _H + i * _L, _L)] = zero16
        xr_v[pl.ds(_N * _H + i * _L, _L)] = zero16
        return c
    lax.fori_loop(0, (_NPAD - _N) * _H // _L, zpad, 0)

    for c in c_small:
        c.wait()
    bias_chunks = [bias_v[pl.ds(fc * _L, _L)] for fc in range(_H // _L)]

    def binit(row, c):
        for fc in range(_H // _L):
            out_v[pl.ds(row * _H + fc * _L, _L)] = bias_chunks[fc]
        return c

    def dzero(i, c):
        den_v[pl.ds(i * _L, _L)] = zero16
        return c

    def run_passes():
        @plsc.parallel_loop(0, _EP // _L, unroll=2,
                            carry=jnp.full((_L,), -1e30, jnp.float32))
        def score_chunk(c, gmax):
            src16 = src_v[pl.ds(c * _L, _L)]
            dst16 = dst_v[pl.ds(c * _L, _L)]
            sb = src16 * _H
            db = dst16 * _H
            def fblock(fo, accs):
                accs = list(accs)
                fb = fo * _L
                for fi in range(_L):
                    rot = iota16 ^ (fb + fi)
                    attf = att_v[pl.ds((fb + fi) * _L, _L)]
                    sl = plsc.load_gather(xl_v, [sb + rot])
                    dl = plsc.load_gather(xr_v, [db + rot])
                    m = sl + dl
                    hh = jnp.where(m >= 0, m, jnp.float32(0.2) * m)
                    accs[fi % 4] = accs[fi % 4] + attf * hh
                return tuple(accs)
            accs = lax.fori_loop(0, _H // _L, fblock,
                                 (zero16, zero16, zero16, zero16))
            acc = (accs[0] + accs[1]) + (accs[2] + accs[3])
            e_v[pl.ds(c * _L, _L)] = acc
            return jnp.maximum(gmax, acc)
        gmax = jnp.max(score_chunk)

        @plsc.parallel_loop(0, _EP // _L, carry=jnp.int32(0))
        def den_chunk(c, carry):
            dst16 = dst_v[pl.ds(c * _L, _L)]
            ex = jnp.exp(e_v[pl.ds(c * _L, _L)] - gmax)
            e_v[pl.ds(c * _L, _L)] = ex
            plsc.addupdate_scatter(den_v, [dst16], ex)
            return carry

        @plsc.parallel_loop(0, _EP // _L, unroll=2, carry=jnp.int32(0))
        def agg_chunk(c, carry):
            src16 = src_v[pl.ds(c * _L, _L)]
            dst16 = dst_v[pl.ds(c * _L, _L)]
            sb = src16 * _H
            db = dst16 * _H
            ex = e_v[pl.ds(c * _L, _L)]
            dn = plsc.load_gather(den_v, [dst16])
            alpha = ex / (dn + jnp.float32(1e-16))
            def ablock(fo, c2):
                fb = fo * _L
                for fi in range(_L):
                    rot = iota16 ^ (fb + fi)
                    xv = plsc.load_gather(xl_v, [sb + rot])
                    plsc.addupdate_scatter(out_v, [db + rot], alpha * xv)
                return c2
            lax.fori_loop(0, _H // _L, ablock, 0)
            return carry

    lax.fori_loop(0, _NPAD, binit, 0)
    lax.fori_loop(0, 320 // _L, dzero, 0)
    c_xl.wait()
    c_xr.wait()
    run_passes()

    c_out = pltpu.async_copy(out_v.at[pl.ds(0, _N * _H)],
                             out_hbm.at[pl.ds(base0, _N * _H)], sem)
    c_xl = pltpu.async_copy(xl_hbm.at[pl.ds(base1, _N * _H)],
                            xl_v.at[pl.ds(0, _N * _H)], sem)
    c_xr = pltpu.async_copy(xr_hbm.at[pl.ds(base1, _N * _H)],
                            xr_v.at[pl.ds(0, _N * _H)], sem)
    lax.fori_loop(0, 320 // _L, dzero, 0)
    c_out.wait()
    lax.fori_loop(0, _NPAD, binit, 0)
    c_xl.wait()
    c_xr.wait()
    run_passes()
    pltpu.sync_copy(out_v.at[pl.ds(0, _N * _H)],
                    out_hbm.at[pl.ds(base1, _N * _H)])


def kernel(t, z, edge_index, Wl, Wr, att, bias):
    h = z.shape[1]
    n = _N
    b = z.shape[0] // n
    e = edge_index.shape[1]
    et = e + n
    loop = jnp.arange(n, dtype=jnp.int32)
    pad = jnp.full((_EP - et,), n, jnp.int32)
    src = jnp.concatenate([edge_index[0].astype(jnp.int32), loop, pad])
    dst = jnp.concatenate([edge_index[1].astype(jnp.int32), loop, pad])

    rows = b * n
    nch = 8
    blk = rows // nch
    xl, xr = pl.pallas_call(
        _proj_body,
        grid=(nch,),
        in_specs=[
            pl.BlockSpec((blk, h), lambda i: (i, 0)),
            pl.BlockSpec((h, h), lambda i: (0, 0)),
            pl.BlockSpec((h, h), lambda i: (0, 0)),
        ],
        out_specs=[
            pl.BlockSpec((blk, h), lambda i: (i, 0)),
            pl.BlockSpec((blk, h), lambda i: (i, 0)),
        ],
        out_shape=[
            jax.ShapeDtypeStruct((rows, h), jnp.float32),
            jax.ShapeDtypeStruct((rows, h), jnp.float32),
        ],
    )(z, Wl, Wr)

    sc = pl.kernel(
        _sc_body,
        out_type=jax.ShapeDtypeStruct((rows * h,), jnp.float32),
        mesh=plsc.VectorSubcoreMesh(core_axis_name="c", subcore_axis_name="s",
                                    num_cores=_NC, num_subcores=_NS),
        compiler_params=pltpu.CompilerParams(needs_layout_passes=False),
        scratch_types=[
            pltpu.VMEM((_NPAD * _H,), jnp.float32),
            pltpu.VMEM((_NPAD * _H,), jnp.float32),
            pltpu.VMEM((_NPAD * _H,), jnp.float32),
            pltpu.VMEM((_EP,), jnp.int32),
            pltpu.VMEM((_EP,), jnp.int32),
            pltpu.VMEM((_EP,), jnp.float32),
            pltpu.VMEM((320,), jnp.float32),
            pltpu.VMEM((_H * _L,), jnp.float32),
            pltpu.VMEM((h,), jnp.float32),
            pltpu.SemaphoreType.DMA,
        ],
    )
    rot_idx = jnp.arange(_H)[:, None] ^ jnp.arange(_L)[None, :]
    att_tab = att[rot_idx].reshape(-1)
    out = sc(xl.reshape(-1), xr.reshape(-1), src, dst, att_tab, bias)
    return out.reshape(rows, h, 1)


# speedup vs baseline: 1.2484x; 1.0003x over previous
import jax
import jax.numpy as jnp
from jax import lax
from jax.experimental import pallas as pl
from jax.experimental.pallas import tpu as pltpu
from jax.experimental.pallas import tpu_sc as plsc

_N = 307
_H = 128
_L = 16
_NC = 2
_NS = 16
_NW = _NC * _NS
_NPAD = 312
_EP = 992


def _proj_body(x_ref, wl_ref, wr_ref, xl_ref, xr_ref):
    x = x_ref[...]
    dn = (((1,), (1,)), ((), ()))
    xl_ref[...] = lax.dot_general(x, wl_ref[...], dn,
                                  preferred_element_type=jnp.float32)
    xr_ref[...] = lax.dot_general(x, wr_ref[...], dn,
                                  preferred_element_type=jnp.float32)


def _sc_body(xl_hbm, xr_hbm, src_hbm, dst_hbm, att_hbm, bias_hbm, out_hbm,
             xl_v, xr_v, out_v, src_v, dst_v, e_v, den_v, att_v, bias_v, sem):
    wid = lax.axis_index("s") * _NC + lax.axis_index("c")
    c_small = [
        pltpu.async_copy(src_hbm, src_v, sem),
        pltpu.async_copy(dst_hbm, dst_v, sem),
        pltpu.async_copy(att_hbm, att_v, sem),
        pltpu.async_copy(bias_hbm, bias_v, sem),
    ]
    base0 = wid * (_N * _H)
    base1 = (wid + _NW) * (_N * _H)
    c_xl = pltpu.async_copy(xl_hbm.at[pl.ds(base0, _N * _H)],
                            xl_v.at[pl.ds(0, _N * _H)], sem)
    c_xr = pltpu.async_copy(xr_hbm.at[pl.ds(base0, _N * _H)],
                            xr_v.at[pl.ds(0, _N * _H)], sem)

    zero16 = jnp.zeros((_L,), jnp.float32)
    iota16 = lax.iota(jnp.int32, _L)

    def zpad(i, c):
        xl_v[pl.ds(_N * _H + i * _L, _L)] = zero16
        xr_v[pl.ds(_N * _H + i * _L, _L)] = zero16
        return c
    lax.fori_loop(0, (_NPAD - _N) * _H // _L, zpad, 0)

    for c in c_small:
        c.wait()
    bias_chunks = [bias_v[pl.ds(fc * _L, _L)] for fc in range(_H // _L)]

    def binit(row, c):
        for fc in range(_H // _L):
            out_v[pl.ds(row * _H + fc * _L, _L)] = bias_chunks[fc]
        return c

    def dzero(i, c):
        den_v[pl.ds(i * _L, _L)] = zero16
        return c

    def run_passes():
        @plsc.parallel_loop(0, _EP // _L, unroll=4,
                            carry=jnp.full((_L,), -1e30, jnp.float32))
        def score_chunk(c, gmax):
            src16 = src_v[pl.ds(c * _L, _L)]
            dst16 = dst_v[pl.ds(c * _L, _L)]
            sb = src16 * _H
            db = dst16 * _H
            def fblock(fo, accs):
                accs = list(accs)
                fb = fo * _L
                for fi in range(_L):
                    rot = iota16 ^ (fb + fi)
                    attf = att_v[pl.ds((fb + fi) * _L, _L)]
                    sl = plsc.load_gather(xl_v, [sb + rot])
                    dl = plsc.load_gather(xr_v, [db + rot])
                    m = sl + dl
                    hh = jnp.where(m >= 0, m, jnp.float32(0.2) * m)
                    accs[fi % 4] = accs[fi % 4] + attf * hh
                return tuple(accs)
            accs = lax.fori_loop(0, _H // _L, fblock,
                                 (zero16, zero16, zero16, zero16))
            acc = (accs[0] + accs[1]) + (accs[2] + accs[3])
            e_v[pl.ds(c * _L, _L)] = acc
            return jnp.maximum(gmax, acc)
        gmax = jnp.max(score_chunk)

        @plsc.parallel_loop(0, _EP // _L, carry=jnp.int32(0))
        def den_chunk(c, carry):
            dst16 = dst_v[pl.ds(c * _L, _L)]
            ex = jnp.exp(e_v[pl.ds(c * _L, _L)] - gmax)
            e_v[pl.ds(c * _L, _L)] = ex
            plsc.addupdate_scatter(den_v, [dst16], ex)
            return carry

        @plsc.parallel_loop(0, _EP // _L, unroll=4, carry=jnp.int32(0))
        def agg_chunk(c, carry):
            src16 = src_v[pl.ds(c * _L, _L)]
            dst16 = dst_v[pl.ds(c * _L, _L)]
            sb = src16 * _H
            db = dst16 * _H
            ex = e_v[pl.ds(c * _L, _L)]
            dn = plsc.load_gather(den_v, [dst16])
            alpha = ex / (dn + jnp.float32(1e-16))
            def ablock(fo, c2):
                fb = fo * _L
                for fi in range(_L):
                    rot = iota16 ^ (fb + fi)
                    xv = plsc.load_gather(xl_v, [sb + rot])
                    plsc.addupdate_scatter(out_v, [db + rot], alpha * xv)
                return c2
            lax.fori_loop(0, _H // _L, ablock, 0)
            return carry

    lax.fori_loop(0, _NPAD, binit, 0)
    lax.fori_loop(0, 320 // _L, dzero, 0)
    c_xl.wait()
    c_xr.wait()
    run_passes()

    c_out = pltpu.async_copy(out_v.at[pl.ds(0, _N * _H)],
                             out_hbm.at[pl.ds(base0, _N * _H)], sem)
    c_xl = pltpu.async_copy(xl_hbm.at[pl.ds(base1, _N * _H)],
                            xl_v.at[pl.ds(0, _N * _H)], sem)
    c_xr = pltpu.async_copy(xr_hbm.at[pl.ds(base1, _N * _H)],
                            xr_v.at[pl.ds(0, _N * _H)], sem)
    lax.fori_loop(0, 320 // _L, dzero, 0)
    c_out.wait()
    lax.fori_loop(0, _NPAD, binit, 0)
    c_xl.wait()
    c_xr.wait()
    run_passes()
    pltpu.sync_copy(out_v.at[pl.ds(0, _N * _H)],
                    out_hbm.at[pl.ds(base1, _N * _H)])


def kernel(t, z, edge_index, Wl, Wr, att, bias):
    h = z.shape[1]
    n = _N
    b = z.shape[0] // n
    e = edge_index.shape[1]
    et = e + n
    loop = jnp.arange(n, dtype=jnp.int32)
    pad = jnp.full((_EP - et,), n, jnp.int32)
    src = jnp.concatenate([edge_index[0].astype(jnp.int32), loop, pad])
    dst = jnp.concatenate([edge_index[1].astype(jnp.int32), loop, pad])

    rows = b * n
    nch = 8
    blk = rows // nch
    xl, xr = pl.pallas_call(
        _proj_body,
        grid=(nch,),
        in_specs=[
            pl.BlockSpec((blk, h), lambda i: (i, 0)),
            pl.BlockSpec((h, h), lambda i: (0, 0)),
            pl.BlockSpec((h, h), lambda i: (0, 0)),
        ],
        out_specs=[
            pl.BlockSpec((blk, h), lambda i: (i, 0)),
            pl.BlockSpec((blk, h), lambda i: (i, 0)),
        ],
        out_shape=[
            jax.ShapeDtypeStruct((rows, h), jnp.float32),
            jax.ShapeDtypeStruct((rows, h), jnp.float32),
        ],
    )(z, Wl, Wr)

    sc = pl.kernel(
        _sc_body,
        out_type=jax.ShapeDtypeStruct((rows * h,), jnp.float32),
        mesh=plsc.VectorSubcoreMesh(core_axis_name="c", subcore_axis_name="s",
                                    num_cores=_NC, num_subcores=_NS),
        compiler_params=pltpu.CompilerParams(needs_layout_passes=False),
        scratch_types=[
            pltpu.VMEM((_NPAD * _H,), jnp.float32),
            pltpu.VMEM((_NPAD * _H,), jnp.float32),
            pltpu.VMEM((_NPAD * _H,), jnp.float32),
            pltpu.VMEM((_EP,), jnp.int32),
            pltpu.VMEM((_EP,), jnp.int32),
            pltpu.VMEM((_EP,), jnp.float32),
            pltpu.VMEM((320,), jnp.float32),
            pltpu.VMEM((_H * _L,), jnp.float32),
            pltpu.VMEM((h,), jnp.float32),
            pltpu.SemaphoreType.DMA,
        ],
    )
    rot_idx = jnp.arange(_H)[:, None] ^ jnp.arange(_L)[None, :]
    att_tab = att[rot_idx].reshape(-1)
    out = sc(xl.reshape(-1), xr.reshape(-1), src, dst, att_tab, bias)
    return out.reshape(rows, h, 1)


# speedup vs baseline: 1.2504x; 1.0016x over previous
import jax
import jax.numpy as jnp
from jax import lax
from jax.experimental import pallas as pl
from jax.experimental.pallas import tpu as pltpu
from jax.experimental.pallas import tpu_sc as plsc

_N = 307
_H = 128
_L = 16
_NC = 2
_NS = 16
_NW = _NC * _NS
_NPAD = 312
_EP = 992


def _proj_body(x_ref, wl_ref, wr_ref, xl_ref, xr_ref):
    x = x_ref[...]
    dn = (((1,), (1,)), ((), ()))
    xl_ref[...] = lax.dot_general(x, wl_ref[...], dn,
                                  preferred_element_type=jnp.float32)
    xr_ref[...] = lax.dot_general(x, wr_ref[...], dn,
                                  preferred_element_type=jnp.float32)


def _sc_body(xl_hbm, xr_hbm, src_hbm, dst_hbm, att_hbm, bias_hbm, out_hbm,
             xl_v, xr_v, out_v, src_v, dst_v, e_v, den_v, att_v, bias_v, sem):
    wid = lax.axis_index("s") * _NC + lax.axis_index("c")
    c_small = [
        pltpu.async_copy(src_hbm, src_v, sem),
        pltpu.async_copy(dst_hbm, dst_v, sem),
        pltpu.async_copy(att_hbm, att_v, sem),
        pltpu.async_copy(bias_hbm, bias_v, sem),
    ]
    base0 = wid * (_N * _H)
    base1 = (wid + _NW) * (_N * _H)
    c_xl = pltpu.async_copy(xl_hbm.at[pl.ds(base0, _N * _H)],
                            xl_v.at[pl.ds(0, _N * _H)], sem)
    c_xr = pltpu.async_copy(xr_hbm.at[pl.ds(base0, _N * _H)],
                            xr_v.at[pl.ds(0, _N * _H)], sem)

    zero16 = jnp.zeros((_L,), jnp.float32)
    iota16 = lax.iota(jnp.int32, _L)

    def zpad(i, c):
        xl_v[pl.ds(_N * _H + i * _L, _L)] = zero16
        xr_v[pl.ds(_N * _H + i * _L, _L)] = zero16
        return c
    lax.fori_loop(0, (_NPAD - _N) * _H // _L, zpad, 0)

    for c in c_small:
        c.wait()
    bias_chunks = [bias_v[pl.ds(fc * _L, _L)] for fc in range(_H // _L)]

    def binit(row, c):
        for fc in range(_H // _L):
            out_v[pl.ds(row * _H + fc * _L, _L)] = bias_chunks[fc]
        return c

    def dzero(i, c):
        den_v[pl.ds(i * _L, _L)] = zero16
        return c

    def run_passes():
        @plsc.parallel_loop(0, _EP // _L, unroll=2,
                            carry=jnp.full((_L,), -1e30, jnp.float32))
        def score_chunk(c, gmax):
            src16 = src_v[pl.ds(c * _L, _L)]
            dst16 = dst_v[pl.ds(c * _L, _L)]
            sb = src16 * _H
            db = dst16 * _H
            def fblock(fo, accs):
                accs = list(accs)
                fb = fo * _L
                for fi in range(_L):
                    rot = iota16 ^ (fb + fi)
                    attf = att_v[pl.ds((fb + fi) * _L, _L)]
                    sl = plsc.load_gather(xl_v, [sb + rot])
                    dl = plsc.load_gather(xr_v, [db + rot])
                    m = sl + dl
                    hh = jnp.where(m >= 0, m, jnp.float32(0.2) * m)
                    accs[fi % 4] = accs[fi % 4] + attf * hh
                return tuple(accs)
            accs = lax.fori_loop(0, _H // _L, fblock,
                                 (zero16, zero16, zero16, zero16))
            acc = (accs[0] + accs[1]) + (accs[2] + accs[3])
            e_v[pl.ds(c * _L, _L)] = acc
            return jnp.maximum(gmax, acc)
        gmax = jnp.max(score_chunk)

        @plsc.parallel_loop(0, _EP // _L, carry=jnp.int32(0))
        def den_chunk(c, carry):
            dst16 = dst_v[pl.ds(c * _L, _L)]
            ex = jnp.exp(e_v[pl.ds(c * _L, _L)] - gmax)
            e_v[pl.ds(c * _L, _L)] = ex
            plsc.addupdate_scatter(den_v, [dst16], ex)
            return carry

        @plsc.parallel_loop(0, _EP // _L, unroll=2, carry=jnp.int32(0))
        def agg_chunk(c, carry):
            src16 = src_v[pl.ds(c * _L, _L)]
            dst16 = dst_v[pl.ds(c * _L, _L)]
            sb = src16 * _H
            db = dst16 * _H
            ex = e_v[pl.ds(c * _L, _L)]
            dn = plsc.load_gather(den_v, [dst16])
            alpha = ex / (dn + jnp.float32(1e-16))
            def ablock(fo, c2):
                fb = fo * _L
                for fi in range(_L):
                    rot = iota16 ^ (fb + fi)
                    xv = plsc.load_gather(xl_v, [sb + rot])
                    plsc.addupdate_scatter(out_v, [db + rot], alpha * xv)
                return c2
            lax.fori_loop(0, _H // _L, ablock, 0)
            return carry

    lax.fori_loop(0, _NPAD, binit, 0)
    lax.fori_loop(0, 320 // _L, dzero, 0)
    c_xl.wait()
    c_xr.wait()
    run_passes()

    c_out = pltpu.async_copy(out_v.at[pl.ds(0, _N * _H)],
                             out_hbm.at[pl.ds(base0, _N * _H)], sem)
    c_xl = pltpu.async_copy(xl_hbm.at[pl.ds(base1, _N * _H)],
                            xl_v.at[pl.ds(0, _N * _H)], sem)
    c_xr = pltpu.async_copy(xr_hbm.at[pl.ds(base1, _N * _H)],
                            xr_v.at[pl.ds(0, _N * _H)], sem)
    lax.fori_loop(0, 320 // _L, dzero, 0)
    c_out.wait()
    lax.fori_loop(0, _NPAD, binit, 0)
    c_xl.wait()
    c_xr.wait()
    run_passes()
    pltpu.sync_copy(out_v.at[pl.ds(0, _N * _H)],
                    out_hbm.at[pl.ds(base1, _N * _H)])


def kernel(t, z, edge_index, Wl, Wr, att, bias):
    h = z.shape[1]
    n = _N
    b = z.shape[0] // n
    e = edge_index.shape[1]
    et = e + n
    loop = jnp.arange(n, dtype=jnp.int32)
    pad = jnp.full((_EP - et,), n, jnp.int32)
    src = jnp.concatenate([edge_index[0].astype(jnp.int32), loop, pad])
    dst = jnp.concatenate([edge_index[1].astype(jnp.int32), loop, pad])

    rows = b * n
    nch = 8
    blk = rows // nch
    xl, xr = pl.pallas_call(
        _proj_body,
        grid=(nch,),
        in_specs=[
            pl.BlockSpec((blk, h), lambda i: (i, 0)),
            pl.BlockSpec((h, h), lambda i: (0, 0)),
            pl.BlockSpec((h, h), lambda i: (0, 0)),
        ],
        out_specs=[
            pl.BlockSpec((blk, h), lambda i: (i, 0)),
            pl.BlockSpec((blk, h), lambda i: (i, 0)),
        ],
        out_shape=[
            jax.ShapeDtypeStruct((rows, h), jnp.float32),
            jax.ShapeDtypeStruct((rows, h), jnp.float32),
        ],
    )(z, Wl, Wr)

    sc = pl.kernel(
        _sc_body,
        out_type=jax.ShapeDtypeStruct((rows * h,), jnp.float32),
        mesh=plsc.VectorSubcoreMesh(core_axis_name="c", subcore_axis_name="s",
                                    num_cores=_NC, num_subcores=_NS),
        compiler_params=pltpu.CompilerParams(needs_layout_passes=False),
        scratch_types=[
            pltpu.VMEM((_NPAD * _H,), jnp.float32),
            pltpu.VMEM((_NPAD * _H,), jnp.float32),
            pltpu.VMEM((_NPAD * _H,), jnp.float32),
            pltpu.VMEM((_EP,), jnp.int32),
            pltpu.VMEM((_EP,), jnp.int32),
            pltpu.VMEM((_EP,), jnp.float32),
            pltpu.VMEM((320,), jnp.float32),
            pltpu.VMEM((_H * _L,), jnp.float32),
            pltpu.VMEM((h,), jnp.float32),
            pltpu.SemaphoreType.DMA,
        ],
    )
    rot_idx = jnp.arange(_H)[:, None] ^ jnp.arange(_L)[None, :]
    att_tab = att[rot_idx].reshape(-1)
    out = sc(xl.reshape(-1), xr.reshape(-1), src, dst, att_tab, bias)
    return out.reshape(rows, h, 1)
